# Initial kernel scaffold; baseline (speedup 1.0000x reference)
import jax
import jax.numpy as jnp
from jax.experimental import pallas as pl


def kernel(x, codebook, ema_count, ema_sum):
    raise NotImplementedError("write your pallas kernel here")


# speedup vs baseline: 1.7383x; 1.7383x over previous
import functools

import jax
import jax.numpy as jnp
from jax.experimental import pallas as pl
from jax.experimental.pallas import tpu as pltpu

N = 16384
K = 1024
D = 64
R = 1024
BETA = 0.25
DECAY = 0.99
USAGE_LAMBDA = 0.005


def _vq_body(x_ref, cb_ref, emac_ref, emas_ref, r_ref,
             quant_ref, idx_ref, loss_ref, ent_ref, perp_ref, counts_ref,
             newcb_ref, newcount_ref, newsum_ref,
             counts_acc, sumvec_acc, sse_acc):
    i = pl.program_id(0)

    @pl.when(i == 0)
    def _init():
        counts_acc[...] = jnp.zeros_like(counts_acc)
        sumvec_acc[...] = jnp.zeros_like(sumvec_acc)
        sse_acc[0] = jnp.float32(0.0)

    x = x_ref[...]
    cb = cb_ref[...]
    xn = x / jnp.clip(jnp.sqrt(jnp.sum(x * x, axis=1, keepdims=True)),
                      1e-12, None)
    cbn = cb / jnp.clip(jnp.sqrt(jnp.sum(cb * cb, axis=1, keepdims=True)),
                        1e-12, None)
    sims = jax.lax.dot_general(xn, cbn, (((1,), (1,)), ((), ())),
                               preferred_element_type=jnp.float32)
    dist = 1.0 - sims
    m = jnp.min(dist, axis=1, keepdims=True)
    iota = jax.lax.broadcasted_iota(jnp.int32, (R, K), 1)
    idxmat = jnp.where(dist == m, iota, jnp.int32(K))
    idx_col = jnp.min(idxmat, axis=1, keepdims=True)
    one_hot = (iota == idx_col).astype(jnp.float32)
    quant = jax.lax.dot_general(one_hot, cb, (((1,), (0,)), ((), ())),
                                preferred_element_type=jnp.float32)
    quant_ref[...] = quant
    idx_ref[0, 0, :] = jnp.min(idxmat, axis=1)

    ones_col = jnp.ones((R, 1), dtype=jnp.float32)
    counts_acc[...] += jax.lax.dot_general(
        one_hot, ones_col, (((0,), (0,)), ((), ())),
        preferred_element_type=jnp.float32)
    sumvec_acc[...] += jax.lax.dot_general(
        one_hot, x, (((0,), (0,)), ((), ())),
        preferred_element_type=jnp.float32)
    diff = x - quant
    sse_acc[0] += jnp.sum(diff * diff)

    @pl.when(i == pl.num_programs(0) - 1)
    def _finalize():
        counts = counts_acc[...]
        total = jnp.sum(counts)
        p = counts / (total + 1e-6)
        ent = -jnp.sum(p * jnp.log(p + 1e-12))
        logC = jnp.log(jnp.float32(K) + 1e-12)
        commit = BETA * sse_acc[0] / jnp.float32(N * D)
        loss_ref[...] = jnp.full((1, 1), commit + USAGE_LAMBDA * (logC - ent),
                                 dtype=jnp.float32)
        ent_ref[...] = jnp.full((1, 1), ent, dtype=jnp.float32)
        perp_ref[...] = jnp.full((1, 1), jnp.exp(ent), dtype=jnp.float32)
        counts_ref[...] = counts
        new_count = emac_ref[...] * DECAY + counts * (1.0 - DECAY)
        new_sum = emas_ref[...] * DECAY + sumvec_acc[...] * (1.0 - DECAY)
        n = new_count + 1e-5
        new_cb = new_sum / n
        dead = new_count < 0.001
        r = r_ref[...]
        newcb_ref[...] = jnp.where(dead, r, new_cb)
        newsum_ref[...] = jnp.where(dead, r, new_sum)
        newcount_ref[...] = jnp.where(dead, jnp.float32(1.0), new_count)


@functools.partial(jax.jit)
def kernel(x, codebook, ema_count, ema_sum):
    grid = N // R
    r = jax.random.normal(jax.random.key(1), (K, D), dtype=jnp.float32)
    rn = jnp.linalg.norm(r, axis=-1, keepdims=True)
    r = r / jnp.clip(rn, 1e-12, None) * 0.1

    out_shapes = (
        jax.ShapeDtypeStruct((N, D), jnp.float32),
        jax.ShapeDtypeStruct((grid, 1, R), jnp.int32),
        jax.ShapeDtypeStruct((1, 1), jnp.float32),
        jax.ShapeDtypeStruct((1, 1), jnp.float32),
        jax.ShapeDtypeStruct((1, 1), jnp.float32),
        jax.ShapeDtypeStruct((K, 1), jnp.float32),
        jax.ShapeDtypeStruct((K, D), jnp.float32),
        jax.ShapeDtypeStruct((K, 1), jnp.float32),
        jax.ShapeDtypeStruct((K, D), jnp.float32),
    )
    in_specs = [
        pl.BlockSpec((R, D), lambda i: (i, 0)),
        pl.BlockSpec((K, D), lambda i: (0, 0)),
        pl.BlockSpec((K, 1), lambda i: (0, 0)),
        pl.BlockSpec((K, D), lambda i: (0, 0)),
        pl.BlockSpec((K, D), lambda i: (0, 0)),
    ]
    out_specs = (
        pl.BlockSpec((R, D), lambda i: (i, 0)),
        pl.BlockSpec((1, 1, R), lambda i: (i, 0, 0)),
        pl.BlockSpec((1, 1), lambda i: (0, 0)),
        pl.BlockSpec((1, 1), lambda i: (0, 0)),
        pl.BlockSpec((1, 1), lambda i: (0, 0)),
        pl.BlockSpec((K, 1), lambda i: (0, 0)),
        pl.BlockSpec((K, D), lambda i: (0, 0)),
        pl.BlockSpec((K, 1), lambda i: (0, 0)),
        pl.BlockSpec((K, D), lambda i: (0, 0)),
    )
    outs = pl.pallas_call(
        _vq_body,
        grid=(grid,),
        in_specs=in_specs,
        out_specs=out_specs,
        out_shape=out_shapes,
        scratch_shapes=[
            pltpu.VMEM((K, 1), jnp.float32),
            pltpu.VMEM((K, D), jnp.float32),
            pltpu.SMEM((1,), jnp.float32),
        ],
    )(x, codebook, ema_count.reshape(K, 1), ema_sum, r)
    (quant, idx3, vq, ent, perp, counts, new_cb, new_count, new_sum) = outs
    return (quant,
            idx3.reshape(N),
            vq[0, 0],
            ent[0, 0],
            perp[0, 0],
            counts.reshape(K),
            new_cb,
            new_count.reshape(K),
            new_sum)


# speedup vs baseline: 2.1438x; 1.2333x over previous
import functools

import jax
import jax.numpy as jnp
from jax.experimental import pallas as pl
from jax.experimental.pallas import tpu as pltpu

N = 16384
K = 1024
D = 64
R = 1024
BETA = 0.25
DECAY = 0.99
USAGE_LAMBDA = 0.005


def _vq_body(x_ref, cb_ref, emac_ref, emas_ref, r_ref,
             quant_ref, idx_ref, loss_ref, ent_ref, perp_ref, counts_ref,
             newcb_ref, newcount_ref, newsum_ref,
             sumvec_acc, cbn_scratch, sse_acc):
    i = pl.program_id(0)

    @pl.when(i == 0)
    def _init():
        sumvec_acc[...] = jnp.zeros_like(sumvec_acc)
        sse_acc[0] = jnp.float32(0.0)
        cb0 = cb_ref[...]
        cbn_scratch[...] = cb0 / jnp.clip(
            jnp.sqrt(jnp.sum(cb0 * cb0, axis=1, keepdims=True)), 1e-12, None)

    x = x_ref[...]
    cb = cb_ref[...]
    xn = x / jnp.clip(jnp.sqrt(jnp.sum(x * x, axis=1, keepdims=True)),
                      1e-12, None)
    cbn = cbn_scratch[...]
    sims = jax.lax.dot_general(xn, cbn, (((1,), (1,)), ((), ())),
                               preferred_element_type=jnp.float32)
    dist = 1.0 - sims
    idx = jnp.argmin(dist, axis=1)
    iota = jax.lax.broadcasted_iota(jnp.int32, (R, K), 1)
    one_hot = (iota == idx[:, None]).astype(jnp.float32)
    quant = jax.lax.dot_general(one_hot, cb, (((1,), (0,)), ((), ())),
                                preferred_element_type=jnp.float32)
    quant_ref[...] = quant
    idx_ref[0, 0, :] = idx

    xa = jnp.concatenate([x, jnp.ones((R, 128 - D), dtype=jnp.float32)],
                         axis=1)
    sumvec_acc[...] += jax.lax.dot_general(
        one_hot, xa, (((0,), (0,)), ((), ())),
        preferred_element_type=jnp.float32)
    diff = x - quant
    sse_acc[0] += jnp.sum(diff * diff)

    @pl.when(i == pl.num_programs(0) - 1)
    def _finalize():
        counts = sumvec_acc[:, D:D + 1]
        total = jnp.sum(counts)
        p = counts / (total + 1e-6)
        ent = -jnp.sum(p * jnp.log(p + 1e-12))
        logC = jnp.log(jnp.float32(K) + 1e-12)
        commit = BETA * sse_acc[0] / jnp.float32(N * D)
        loss_ref[...] = jnp.full((1, 1), commit + USAGE_LAMBDA * (logC - ent),
                                 dtype=jnp.float32)
        ent_ref[...] = jnp.full((1, 1), ent, dtype=jnp.float32)
        perp_ref[...] = jnp.full((1, 1), jnp.exp(ent), dtype=jnp.float32)
        counts_ref[...] = counts
        new_count = emac_ref[...] * DECAY + counts * (1.0 - DECAY)
        new_sum = emas_ref[...] * DECAY + sumvec_acc[:, :D] * (1.0 - DECAY)
        n = new_count + 1e-5
        new_cb = new_sum / n
        dead = new_count < 0.001
        r = r_ref[...]
        newcb_ref[...] = jnp.where(dead, r, new_cb)
        newsum_ref[...] = jnp.where(dead, r, new_sum)
        newcount_ref[...] = jnp.where(dead, jnp.float32(1.0), new_count)


@functools.partial(jax.jit)
def kernel(x, codebook, ema_count, ema_sum):
    grid = N // R
    r = jax.random.normal(jax.random.key(1), (K, D), dtype=jnp.float32)
    rn = jnp.linalg.norm(r, axis=-1, keepdims=True)
    r = r / jnp.clip(rn, 1e-12, None) * 0.1

    out_shapes = (
        jax.ShapeDtypeStruct((N, D), jnp.float32),
        jax.ShapeDtypeStruct((grid, 1, R), jnp.int32),
        jax.ShapeDtypeStruct((1, 1), jnp.float32),
        jax.ShapeDtypeStruct((1, 1), jnp.float32),
        jax.ShapeDtypeStruct((1, 1), jnp.float32),
        jax.ShapeDtypeStruct((K, 1), jnp.float32),
        jax.ShapeDtypeStruct((K, D), jnp.float32),
        jax.ShapeDtypeStruct((K, 1), jnp.float32),
        jax.ShapeDtypeStruct((K, D), jnp.float32),
    )
    in_specs = [
        pl.BlockSpec((R, D), lambda i: (i, 0)),
        pl.BlockSpec((K, D), lambda i: (0, 0)),
        pl.BlockSpec((K, 1), lambda i: (0, 0)),
        pl.BlockSpec((K, D), lambda i: (0, 0)),
        pl.BlockSpec((K, D), lambda i: (0, 0)),
    ]
    out_specs = (
        pl.BlockSpec((R, D), lambda i: (i, 0)),
        pl.BlockSpec((1, 1, R), lambda i: (i, 0, 0)),
        pl.BlockSpec((1, 1), lambda i: (0, 0)),
        pl.BlockSpec((1, 1), lambda i: (0, 0)),
        pl.BlockSpec((1, 1), lambda i: (0, 0)),
        pl.BlockSpec((K, 1), lambda i: (0, 0)),
        pl.BlockSpec((K, D), lambda i: (0, 0)),
        pl.BlockSpec((K, 1), lambda i: (0, 0)),
        pl.BlockSpec((K, D), lambda i: (0, 0)),
    )
    outs = pl.pallas_call(
        _vq_body,
        grid=(grid,),
        in_specs=in_specs,
        out_specs=out_specs,
        out_shape=out_shapes,
        scratch_shapes=[
            pltpu.VMEM((K, 128), jnp.float32),
            pltpu.VMEM((K, D), jnp.float32),
            pltpu.SMEM((1,), jnp.float32),
        ],
    )(x, codebook, ema_count.reshape(K, 1), ema_sum, r)
    (quant, idx3, vq, ent, perp, counts, new_cb, new_count, new_sum) = outs
    return (quant,
            idx3.reshape(N),
            vq[0, 0],
            ent[0, 0],
            perp[0, 0],
            counts.reshape(K),
            new_cb,
            new_count.reshape(K),
            new_sum)


# speedup vs baseline: 2.1831x; 1.0183x over previous
import functools

import jax
import jax.numpy as jnp
from jax.experimental import pallas as pl
from jax.experimental.pallas import tpu as pltpu

N = 16384
K = 1024
D = 64
R = 2048
BETA = 0.25
DECAY = 0.99
USAGE_LAMBDA = 0.005


def _vq_body(x_ref, cb_ref, emac_ref, emas_ref, r_ref,
             quant_ref, idx_ref, loss_ref, ent_ref, perp_ref, counts_ref,
             newcb_ref, newcount_ref, newsum_ref,
             sumvec_acc, cbn_scratch, sse_acc):
    i = pl.program_id(0)

    @pl.when(i == 0)
    def _init():
        sumvec_acc[...] = jnp.zeros_like(sumvec_acc)
        sse_acc[0] = jnp.float32(0.0)
        cb0 = cb_ref[...]
        cbn_scratch[...] = cb0 / jnp.clip(
            jnp.sqrt(jnp.sum(cb0 * cb0, axis=1, keepdims=True)), 1e-12, None)

    x = x_ref[...]
    cb = cb_ref[...]
    xn = x / jnp.clip(jnp.sqrt(jnp.sum(x * x, axis=1, keepdims=True)),
                      1e-12, None)
    cbn = cbn_scratch[...]
    sims = jax.lax.dot_general(xn, cbn, (((1,), (1,)), ((), ())),
                               preferred_element_type=jnp.float32)
    dist = 1.0 - sims
    idx = jnp.argmin(dist, axis=1)
    iota = jax.lax.broadcasted_iota(jnp.int32, (R, K), 1)
    one_hot = (iota == idx[:, None]).astype(jnp.float32)
    quant = jax.lax.dot_general(one_hot, cb, (((1,), (0,)), ((), ())),
                                preferred_element_type=jnp.float32)
    quant_ref[...] = quant
    idx_ref[0, 0, :] = idx

    xa = jnp.concatenate([x, jnp.ones((R, 128 - D), dtype=jnp.float32)],
                         axis=1)
    sumvec_acc[...] += jax.lax.dot_general(
        one_hot, xa, (((0,), (0,)), ((), ())),
        preferred_element_type=jnp.float32)
    diff = x - quant
    sse_acc[0] += jnp.sum(diff * diff)

    @pl.when(i == pl.num_programs(0) - 1)
    def _finalize():
        counts = sumvec_acc[:, D:D + 1]
        total = jnp.sum(counts)
        p = counts / (total + 1e-6)
        ent = -jnp.sum(p * jnp.log(p + 1e-12))
        logC = jnp.log(jnp.float32(K) + 1e-12)
        commit = BETA * sse_acc[0] / jnp.float32(N * D)
        loss_ref[...] = jnp.full((1, 1), commit + USAGE_LAMBDA * (logC - ent),
                                 dtype=jnp.float32)
        ent_ref[...] = jnp.full((1, 1), ent, dtype=jnp.float32)
        perp_ref[...] = jnp.full((1, 1), jnp.exp(ent), dtype=jnp.float32)
        counts_ref[...] = counts
        new_count = emac_ref[...] * DECAY + counts * (1.0 - DECAY)
        new_sum = emas_ref[...] * DECAY + sumvec_acc[:, :D] * (1.0 - DECAY)
        n = new_count + 1e-5
        new_cb = new_sum / n
        dead = new_count < 0.001
        r = r_ref[...]
        newcb_ref[...] = jnp.where(dead, r, new_cb)
        newsum_ref[...] = jnp.where(dead, r, new_sum)
        newcount_ref[...] = jnp.where(dead, jnp.float32(1.0), new_count)


@functools.partial(jax.jit)
def kernel(x, codebook, ema_count, ema_sum):
    grid = N // R
    r = jax.random.normal(jax.random.key(1), (K, D), dtype=jnp.float32)
    rn = jnp.linalg.norm(r, axis=-1, keepdims=True)
    r = r / jnp.clip(rn, 1e-12, None) * 0.1

    out_shapes = (
        jax.ShapeDtypeStruct((N, D), jnp.float32),
        jax.ShapeDtypeStruct((grid, 1, R), jnp.int32),
        jax.ShapeDtypeStruct((1, 1), jnp.float32),
        jax.ShapeDtypeStruct((1, 1), jnp.float32),
        jax.ShapeDtypeStruct((1, 1), jnp.float32),
        jax.ShapeDtypeStruct((K, 1), jnp.float32),
        jax.ShapeDtypeStruct((K, D), jnp.float32),
        jax.ShapeDtypeStruct((K, 1), jnp.float32),
        jax.ShapeDtypeStruct((K, D), jnp.float32),
    )
    in_specs = [
        pl.BlockSpec((R, D), lambda i: (i, 0)),
        pl.BlockSpec((K, D), lambda i: (0, 0)),
        pl.BlockSpec((K, 1), lambda i: (0, 0)),
        pl.BlockSpec((K, D), lambda i: (0, 0)),
        pl.BlockSpec((K, D), lambda i: (0, 0)),
    ]
    out_specs = (
        pl.BlockSpec((R, D), lambda i: (i, 0)),
        pl.BlockSpec((1, 1, R), lambda i: (i, 0, 0)),
        pl.BlockSpec((1, 1), lambda i: (0, 0)),
        pl.BlockSpec((1, 1), lambda i: (0, 0)),
        pl.BlockSpec((1, 1), lambda i: (0, 0)),
        pl.BlockSpec((K, 1), lambda i: (0, 0)),
        pl.BlockSpec((K, D), lambda i: (0, 0)),
        pl.BlockSpec((K, 1), lambda i: (0, 0)),
        pl.BlockSpec((K, D), lambda i: (0, 0)),
    )
    outs = pl.pallas_call(
        _vq_body,
        grid=(grid,),
        in_specs=in_specs,
        out_specs=out_specs,
        out_shape=out_shapes,
        scratch_shapes=[
            pltpu.VMEM((K, 128), jnp.float32),
            pltpu.VMEM((K, D), jnp.float32),
            pltpu.SMEM((1,), jnp.float32),
        ],
    )(x, codebook, ema_count.reshape(K, 1), ema_sum, r)
    (quant, idx3, vq, ent, perp, counts, new_cb, new_count, new_sum) = outs
    return (quant,
            idx3.reshape(N),
            vq[0, 0],
            ent[0, 0],
            perp[0, 0],
            counts.reshape(K),
            new_cb,
            new_count.reshape(K),
            new_sum)


# speedup vs baseline: 2.3208x; 1.0631x over previous
import functools

import numpy as np

import jax
import jax.numpy as jnp
from jax.experimental import pallas as pl
from jax.experimental.pallas import tpu as pltpu

N = 16384
K = 1024
D = 64
R = 2048
BETA = 0.25
DECAY = 0.99
USAGE_LAMBDA = 0.005


def _reinit_table() -> np.ndarray:
    r = jax.random.normal(jax.random.key(1), (K, D), dtype=jnp.float32)
    rn = jnp.linalg.norm(r, axis=-1, keepdims=True)
    return np.asarray(r / jnp.clip(rn, 1e-12, None) * 0.1)


_R_TABLE = _reinit_table()


def _vq_body(x_ref, cb_ref, emac_ref, emas_ref, r_ref,
             quant_ref, idx_ref, loss_ref, ent_ref, perp_ref, counts_ref,
             newcb_ref, newcount_ref, newsum_ref,
             sumvec_acc, cbn_scratch, sse_acc):
    i = pl.program_id(0)

    @pl.when(i == 0)
    def _init():
        sumvec_acc[...] = jnp.zeros_like(sumvec_acc)
        sse_acc[0] = jnp.float32(0.0)
        cb0 = cb_ref[...]
        cbn_scratch[...] = cb0 / jnp.clip(
            jnp.sqrt(jnp.sum(cb0 * cb0, axis=1, keepdims=True)), 1e-12, None)

    x = x_ref[...]
    cb = cb_ref[...]
    xn = x / jnp.clip(jnp.sqrt(jnp.sum(x * x, axis=1, keepdims=True)),
                      1e-12, None)
    cbn = cbn_scratch[...]
    sims = jax.lax.dot_general(xn, cbn, (((1,), (1,)), ((), ())),
                               preferred_element_type=jnp.float32)
    dist = 1.0 - sims
    idx = jnp.argmin(dist, axis=1)
    iota = jax.lax.broadcasted_iota(jnp.int32, (R, K), 1)
    one_hot = (iota == idx[:, None]).astype(jnp.float32)
    quant = jax.lax.dot_general(one_hot, cb, (((1,), (0,)), ((), ())),
                                preferred_element_type=jnp.float32)
    quant_ref[...] = quant
    idx_ref[0, 0, :] = idx

    xa = jnp.concatenate([x, jnp.ones((R, 128 - D), dtype=jnp.float32)],
                         axis=1)
    sumvec_acc[...] += jax.lax.dot_general(
        one_hot, xa, (((0,), (0,)), ((), ())),
        preferred_element_type=jnp.float32)
    diff = x - quant
    sse_acc[0] += jnp.sum(diff * diff)

    @pl.when(i == pl.num_programs(0) - 1)
    def _finalize():
        counts = sumvec_acc[:, D:D + 1]
        total = jnp.sum(counts)
        p = counts / (total + 1e-6)
        ent = -jnp.sum(p * jnp.log(p + 1e-12))
        logC = jnp.log(jnp.float32(K) + 1e-12)
        commit = BETA * sse_acc[0] / jnp.float32(N * D)
        loss_ref[...] = jnp.full((1, 1), commit + USAGE_LAMBDA * (logC - ent),
                                 dtype=jnp.float32)
        ent_ref[...] = jnp.full((1, 1), ent, dtype=jnp.float32)
        perp_ref[...] = jnp.full((1, 1), jnp.exp(ent), dtype=jnp.float32)
        counts_ref[...] = counts
        new_count = emac_ref[...] * DECAY + counts * (1.0 - DECAY)
        new_sum = emas_ref[...] * DECAY + sumvec_acc[:, :D] * (1.0 - DECAY)
        n = new_count + 1e-5
        new_cb = new_sum / n
        dead = new_count < 0.001
        r = r_ref[...]
        newcb_ref[...] = jnp.where(dead, r, new_cb)
        newsum_ref[...] = jnp.where(dead, r, new_sum)
        newcount_ref[...] = jnp.where(dead, jnp.float32(1.0), new_count)


@functools.partial(jax.jit)
def kernel(x, codebook, ema_count, ema_sum):
    grid = N // R
    r = jnp.asarray(_R_TABLE)

    out_shapes = (
        jax.ShapeDtypeStruct((N, D), jnp.float32),
        jax.ShapeDtypeStruct((grid, 1, R), jnp.int32),
        jax.ShapeDtypeStruct((1, 1), jnp.float32),
        jax.ShapeDtypeStruct((1, 1), jnp.float32),
        jax.ShapeDtypeStruct((1, 1), jnp.float32),
        jax.ShapeDtypeStruct((K, 1), jnp.float32),
        jax.ShapeDtypeStruct((K, D), jnp.float32),
        jax.ShapeDtypeStruct((K, 1), jnp.float32),
        jax.ShapeDtypeStruct((K, D), jnp.float32),
    )
    in_specs = [
        pl.BlockSpec((R, D), lambda i: (i, 0)),
        pl.BlockSpec((K, D), lambda i: (0, 0)),
        pl.BlockSpec((K, 1), lambda i: (0, 0)),
        pl.BlockSpec((K, D), lambda i: (0, 0)),
        pl.BlockSpec((K, D), lambda i: (0, 0)),
    ]
    out_specs = (
        pl.BlockSpec((R, D), lambda i: (i, 0)),
        pl.BlockSpec((1, 1, R), lambda i: (i, 0, 0)),
        pl.BlockSpec((1, 1), lambda i: (0, 0)),
        pl.BlockSpec((1, 1), lambda i: (0, 0)),
        pl.BlockSpec((1, 1), lambda i: (0, 0)),
        pl.BlockSpec((K, 1), lambda i: (0, 0)),
        pl.BlockSpec((K, D), lambda i: (0, 0)),
        pl.BlockSpec((K, 1), lambda i: (0, 0)),
        pl.BlockSpec((K, D), lambda i: (0, 0)),
    )
    outs = pl.pallas_call(
        _vq_body,
        grid=(grid,),
        in_specs=in_specs,
        out_specs=out_specs,
        out_shape=out_shapes,
        scratch_shapes=[
            pltpu.VMEM((K, 128), jnp.float32),
            pltpu.VMEM((K, D), jnp.float32),
            pltpu.SMEM((1,), jnp.float32),
        ],
    )(x, codebook, ema_count.reshape(K, 1), ema_sum, r)
    (quant, idx3, vq, ent, perp, counts, new_cb, new_count, new_sum) = outs
    return (quant,
            idx3.reshape(N),
            vq[0, 0],
            ent[0, 0],
            perp[0, 0],
            counts.reshape(K),
            new_cb,
            new_count.reshape(K),
            new_sum)


# speedup vs baseline: 2.4459x; 1.0539x over previous
import functools

import numpy as np

import jax
import jax.numpy as jnp
from jax.experimental import pallas as pl
from jax.experimental.pallas import tpu as pltpu

N = 16384
K = 1024
D = 64
R = 2048
BETA = 0.25
DECAY = 0.99
USAGE_LAMBDA = 0.005


def _make_reinit_table():
    r = jax.random.normal(jax.random.key(1), (K, D), dtype=jnp.float32)
    rn = jnp.linalg.norm(r, axis=-1, keepdims=True)
    return r / jnp.clip(rn, 1e-12, None) * 0.1


try:
    with jax.default_device(jax.local_devices(backend="cpu")[0]):
        _R_TABLE = np.asarray(_make_reinit_table())
except Exception:
    _R_TABLE = None


def _vq_body(x_ref, cb_ref, emac_ref, emas_ref, r_ref,
             quant_ref, idx_ref, loss_ref, ent_ref, perp_ref, counts_ref,
             newcb_ref, newcount_ref, newsum_ref,
             sumvec_acc, cbn_scratch, sse_acc):
    i = pl.program_id(0)

    @pl.when(i == 0)
    def _init():
        sumvec_acc[...] = jnp.zeros_like(sumvec_acc)
        sse_acc[0] = jnp.float32(0.0)
        cb0 = cb_ref[...]
        cbn_scratch[...] = cb0 / jnp.clip(
            jnp.sqrt(jnp.sum(cb0 * cb0, axis=1, keepdims=True)), 1e-12, None)

    x = x_ref[...]
    cb = cb_ref[...]
    xn = x / jnp.clip(jnp.sqrt(jnp.sum(x * x, axis=1, keepdims=True)),
                      1e-12, None)
    cbn = cbn_scratch[...]
    sims = jax.lax.dot_general(xn, cbn, (((1,), (1,)), ((), ())),
                               preferred_element_type=jnp.float32)
    dist = 1.0 - sims
    idx = jnp.argmin(dist, axis=1)
    iota = jax.lax.broadcasted_iota(jnp.int32, (R, K), 1)
    one_hot = (iota == idx[:, None]).astype(jnp.float32)
    quant = jax.lax.dot_general(one_hot, cb, (((1,), (0,)), ((), ())),
                                preferred_element_type=jnp.float32)
    quant_ref[...] = quant
    idx_ref[...] = idx

    xa = jnp.concatenate([x, jnp.ones((R, 128 - D), dtype=jnp.float32)],
                         axis=1)
    sumvec_acc[...] += jax.lax.dot_general(
        one_hot, xa, (((0,), (0,)), ((), ())),
        preferred_element_type=jnp.float32)
    diff = x - quant
    sse_acc[0] += jnp.sum(diff * diff)

    @pl.when(i == pl.num_programs(0) - 1)
    def _finalize():
        counts_col = sumvec_acc[:, D:D + 1]
        counts_lane = jnp.transpose(counts_col).reshape(K)
        total = jnp.sum(counts_lane)
        p = counts_lane / (total + 1e-6)
        ent = -jnp.sum(p * jnp.log(p + 1e-12))
        logC = jnp.log(jnp.float32(K) + 1e-12)
        commit = BETA * sse_acc[0] / jnp.float32(N * D)
        loss_ref[...] = jnp.full((1, 1), commit + USAGE_LAMBDA * (logC - ent),
                                 dtype=jnp.float32)
        ent_ref[...] = jnp.full((1, 1), ent, dtype=jnp.float32)
        perp_ref[...] = jnp.full((1, 1), jnp.exp(ent), dtype=jnp.float32)
        counts_ref[...] = counts_lane
        emac_col = jnp.transpose(emac_ref[...].reshape(1, K))
        new_count = emac_col * DECAY + counts_col * (1.0 - DECAY)
        new_sum = emas_ref[...] * DECAY + sumvec_acc[:, :D] * (1.0 - DECAY)
        n = new_count + 1e-5
        new_cb = new_sum / n
        dead = new_count < 0.001
        r = r_ref[...]
        newcb_ref[...] = jnp.where(dead, r, new_cb)
        newsum_ref[...] = jnp.where(dead, r, new_sum)
        newcount_col = jnp.where(dead, jnp.float32(1.0), new_count)
        newcount_ref[...] = jnp.transpose(newcount_col).reshape(K)


@functools.partial(jax.jit)
def kernel(x, codebook, ema_count, ema_sum):
    grid = N // R
    r = (_make_reinit_table() if _R_TABLE is None else jnp.asarray(_R_TABLE))

    out_shapes = (
        jax.ShapeDtypeStruct((N, D), jnp.float32),
        jax.ShapeDtypeStruct((N,), jnp.int32),
        jax.ShapeDtypeStruct((1, 1), jnp.float32),
        jax.ShapeDtypeStruct((1, 1), jnp.float32),
        jax.ShapeDtypeStruct((1, 1), jnp.float32),
        jax.ShapeDtypeStruct((K,), jnp.float32),
        jax.ShapeDtypeStruct((K, D), jnp.float32),
        jax.ShapeDtypeStruct((K,), jnp.float32),
        jax.ShapeDtypeStruct((K, D), jnp.float32),
    )
    in_specs = [
        pl.BlockSpec((R, D), lambda i: (i, 0)),
        pl.BlockSpec((K, D), lambda i: (0, 0)),
        pl.BlockSpec((K,), lambda i: (0,)),
        pl.BlockSpec((K, D), lambda i: (0, 0)),
        pl.BlockSpec((K, D), lambda i: (0, 0)),
    ]
    out_specs = (
        pl.BlockSpec((R, D), lambda i: (i, 0)),
        pl.BlockSpec((R,), lambda i: (i,)),
        pl.BlockSpec((1, 1), lambda i: (0, 0)),
        pl.BlockSpec((1, 1), lambda i: (0, 0)),
        pl.BlockSpec((1, 1), lambda i: (0, 0)),
        pl.BlockSpec((K,), lambda i: (0,)),
        pl.BlockSpec((K, D), lambda i: (0, 0)),
        pl.BlockSpec((K,), lambda i: (0,)),
        pl.BlockSpec((K, D), lambda i: (0, 0)),
    )
    outs = pl.pallas_call(
        _vq_body,
        grid=(grid,),
        in_specs=in_specs,
        out_specs=out_specs,
        out_shape=out_shapes,
        scratch_shapes=[
            pltpu.VMEM((K, 128), jnp.float32),
            pltpu.VMEM((K, D), jnp.float32),
            pltpu.SMEM((1,), jnp.float32),
        ],
    )(x, codebook, ema_count, ema_sum, r)
    (quant, idx, vq, ent, perp, counts, new_cb, new_count, new_sum) = outs
    return (quant, idx, vq[0, 0], ent[0, 0], perp[0, 0],
            counts, new_cb, new_count, new_sum)


# speedup vs baseline: 3.8877x; 1.5895x over previous
import functools

import numpy as np

import jax
import jax.numpy as jnp
from jax.experimental import pallas as pl
from jax.experimental.pallas import tpu as pltpu

N = 16384
K = 1024
D = 64
R = 2048
BETA = 0.25
DECAY = 0.99
USAGE_LAMBDA = 0.005


def _make_reinit_table():
    r = jax.random.normal(jax.random.key(1), (K, D), dtype=jnp.float32)
    rn = jnp.linalg.norm(r, axis=-1, keepdims=True)
    return r / jnp.clip(rn, 1e-12, None) * 0.1


try:
    with jax.default_device(jax.local_devices(backend="cpu")[0]):
        _R_TABLE_T = np.asarray(_make_reinit_table()).T.copy()
except Exception:
    _R_TABLE_T = None


def _vq_body(xt_ref, cbt_ref, emac_ref, emast_ref, rt_ref,
             quantt_ref, idx_ref, loss_ref, ent_ref, perp_ref, counts_ref,
             newcbt_ref, newcount_ref, newsumt_ref,
             sumvec_acc, cbn_scratch, sse_acc):
    i = pl.program_id(0)

    @pl.when(i == 0)
    def _init():
        sumvec_acc[...] = jnp.zeros_like(sumvec_acc)
        sse_acc[0] = jnp.float32(0.0)
        cbt = cbt_ref[...]
        cbnt = cbt / jnp.clip(
            jnp.sqrt(jnp.sum(cbt * cbt, axis=0, keepdims=True)), 1e-12, None)
        cbn_scratch[...] = jnp.transpose(cbnt)

    xt = xt_ref[...]
    xnt = xt / jnp.clip(jnp.sqrt(jnp.sum(xt * xt, axis=0, keepdims=True)),
                        1e-12, None)
    sims = jax.lax.dot_general(cbn_scratch[...], xnt,
                               (((1,), (0,)), ((), ())),
                               preferred_element_type=jnp.float32)
    dist = 1.0 - sims
    m = jnp.min(dist, axis=0, keepdims=True)
    iota0 = jax.lax.broadcasted_iota(jnp.int32, (K, R), 0)
    idxmat = jnp.where(dist == m, iota0, jnp.int32(K))
    idx = jnp.min(idxmat, axis=0)
    one_hot = (iota0 == idx[None, :]).astype(jnp.float32)
    quantt = jax.lax.dot_general(cbt_ref[...], one_hot,
                                 (((1,), (0,)), ((), ())),
                                 preferred_element_type=jnp.float32)
    quantt_ref[...] = quantt
    idx_ref[...] = idx

    xat = jnp.concatenate([xt, jnp.ones((8, R), dtype=jnp.float32)],
                          axis=0)
    sumvec_acc[...] += jax.lax.dot_general(
        xat, one_hot, (((1,), (1,)), ((), ())),
        preferred_element_type=jnp.float32)
    diff = xt - quantt
    sse_acc[0] += jnp.sum(diff * diff)

    @pl.when(i == pl.num_programs(0) - 1)
    def _finalize():
        counts_row = sumvec_acc[D:D + 1, :]
        total = jnp.sum(counts_row)
        p = counts_row / (total + 1e-6)
        ent = -jnp.sum(p * jnp.log(p + 1e-12))
        logC = jnp.log(jnp.float32(K) + 1e-12)
        commit = BETA * sse_acc[0] / jnp.float32(N * D)
        loss_ref[...] = jnp.full((1, 1), commit + USAGE_LAMBDA * (logC - ent),
                                 dtype=jnp.float32)
        ent_ref[...] = jnp.full((1, 1), ent, dtype=jnp.float32)
        perp_ref[...] = jnp.full((1, 1), jnp.exp(ent), dtype=jnp.float32)
        counts_ref[...] = counts_row.reshape(K)
        emac_row = emac_ref[...].reshape(1, K)
        new_count = emac_row * DECAY + counts_row * (1.0 - DECAY)
        new_sumt = (emast_ref[...] * DECAY
                    + sumvec_acc[:D, :] * (1.0 - DECAY))
        n = new_count + 1e-5
        new_cbt = new_sumt / n
        dead = new_count < 0.001
        rt = rt_ref[...]
        newcbt_ref[...] = jnp.where(dead, rt, new_cbt)
        newsumt_ref[...] = jnp.where(dead, rt, new_sumt)
        newcount_ref[...] = jnp.where(dead, jnp.float32(1.0),
                                      new_count).reshape(K)


@functools.partial(jax.jit)
def kernel(x, codebook, ema_count, ema_sum):
    grid = N // R
    rt = (jnp.transpose(_make_reinit_table()) if _R_TABLE_T is None
          else jnp.asarray(_R_TABLE_T))

    out_shapes = (
        jax.ShapeDtypeStruct((D, N), jnp.float32),
        jax.ShapeDtypeStruct((N,), jnp.int32),
        jax.ShapeDtypeStruct((1, 1), jnp.float32),
        jax.ShapeDtypeStruct((1, 1), jnp.float32),
        jax.ShapeDtypeStruct((1, 1), jnp.float32),
        jax.ShapeDtypeStruct((K,), jnp.float32),
        jax.ShapeDtypeStruct((D, K), jnp.float32),
        jax.ShapeDtypeStruct((K,), jnp.float32),
        jax.ShapeDtypeStruct((D, K), jnp.float32),
    )
    in_specs = [
        pl.BlockSpec((D, R), lambda i: (0, i)),
        pl.BlockSpec((D, K), lambda i: (0, 0)),
        pl.BlockSpec((K,), lambda i: (0,)),
        pl.BlockSpec((D, K), lambda i: (0, 0)),
        pl.BlockSpec((D, K), lambda i: (0, 0)),
    ]
    out_specs = (
        pl.BlockSpec((D, R), lambda i: (0, i)),
        pl.BlockSpec((R,), lambda i: (i,)),
        pl.BlockSpec((1, 1), lambda i: (0, 0)),
        pl.BlockSpec((1, 1), lambda i: (0, 0)),
        pl.BlockSpec((1, 1), lambda i: (0, 0)),
        pl.BlockSpec((K,), lambda i: (0,)),
        pl.BlockSpec((D, K), lambda i: (0, 0)),
        pl.BlockSpec((K,), lambda i: (0,)),
        pl.BlockSpec((D, K), lambda i: (0, 0)),
    )
    outs = pl.pallas_call(
        _vq_body,
        grid=(grid,),
        in_specs=in_specs,
        out_specs=out_specs,
        out_shape=out_shapes,
        scratch_shapes=[
            pltpu.VMEM((D + 8, K), jnp.float32),
            pltpu.VMEM((K, D), jnp.float32),
            pltpu.SMEM((1,), jnp.float32),
        ],
    )(x.T, codebook.T, ema_count, ema_sum.T, rt)
    (quantt, idx, vq, ent, perp, counts, new_cbt, new_count, new_sumt) = outs
    return (quantt.T, idx, vq[0, 0], ent[0, 0], perp[0, 0],
            counts, new_cbt.T, new_count, new_sumt.T)
